# Initial kernel scaffold; baseline (speedup 1.0000x reference)
import jax
import jax.numpy as jnp
from jax.experimental import pallas as pl


def kernel(x, embeddings):
    raise NotImplementedError("write your pallas kernel here")


# speedup vs baseline: 2.1750x; 2.1750x over previous
import functools

import jax
import jax.numpy as jnp
from jax import lax
from jax.experimental import pallas as pl

NUM_LAYERS = 4
NUM_EMBEDDINGS = 1024
EMBEDDING_DIM = 64
COMMITMENT_COST = 0.25

TILE = 1024


def _rvq_tile(x_ref, emb_ref, q_ref, idx_ref, loss_ref):
    i = pl.program_id(0)

    @pl.when(i == 0)
    def _init():
        loss_ref[...] = jnp.zeros((1, 1), jnp.float32)

    r = x_ref[...]
    qacc = jnp.zeros_like(r)
    loss_acc = jnp.float32(0.0)
    for l in range(NUM_LAYERS):
        emb = emb_ref[l]
        e_norms = jnp.sum(emb * emb, axis=1)
        r_norms = jnp.sum(r * r, axis=1, keepdims=True)
        dots = lax.dot_general(
            r, emb, (((1,), (1,)), ((), ())),
            preferred_element_type=jnp.float32,
        )
        dist = (r_norms + e_norms[None, :]) - 2.0 * dots
        mins = jnp.min(dist, axis=1, keepdims=True)
        jidx = lax.broadcasted_iota(jnp.int32, dist.shape, 1)
        idx = jnp.min(
            jnp.where(dist == mins, jidx, NUM_EMBEDDINGS), axis=1
        )
        onehot = (jidx == idx[:, None]).astype(jnp.float32)
        q = lax.dot_general(
            onehot, emb, (((1,), (0,)), ((), ())),
            preferred_element_type=jnp.float32,
        )
        loss_acc += jnp.sum((q - r) * (q - r))
        q_ste = r + (q - r)
        r = r - q_ste
        qacc = qacc + q_ste
        idx_ref[l, :] = idx
    q_ref[...] = qacc
    loss_ref[...] += loss_acc.reshape(1, 1)


@functools.partial(jax.jit, static_argnames=())
def kernel(x, embeddings):
    B, S, D = x.shape
    n_tokens = B * S
    x_flat = x.reshape(n_tokens, D)
    grid = (n_tokens // TILE,)

    q_flat, idx_lt, loss = pl.pallas_call(
        _rvq_tile,
        grid=grid,
        in_specs=[
            pl.BlockSpec((TILE, D), lambda i: (i, 0)),
            pl.BlockSpec((NUM_LAYERS, NUM_EMBEDDINGS, D), lambda i: (0, 0, 0)),
        ],
        out_specs=[
            pl.BlockSpec((TILE, D), lambda i: (i, 0)),
            pl.BlockSpec((NUM_LAYERS, TILE), lambda i: (0, i)),
            pl.BlockSpec((1, 1), lambda i: (0, 0)),
        ],
        out_shape=[
            jax.ShapeDtypeStruct((n_tokens, D), jnp.float32),
            jax.ShapeDtypeStruct((NUM_LAYERS, n_tokens), jnp.int32),
            jax.ShapeDtypeStruct((1, 1), jnp.float32),
        ],
    )(x_flat, embeddings)

    quantized_out = q_flat.reshape(B, S, D)
    losses = loss[0, 0] * (COMMITMENT_COST / n_tokens / D)
    all_indices = idx_lt.T.reshape(B, S, NUM_LAYERS)
    return quantized_out, losses, all_indices
